# baseline (speedup 1.0000x reference)
import functools

import jax
import jax.numpy as jnp
from jax import lax
from jax.experimental import pallas as pl
from jax.experimental.pallas import tpu as pltpu
from jax.experimental.pallas import tpu_sc as plsc

_NC = 2
_NS = 16
_LANES = 16


def _cbow_sparsecore(indices, emb_table):
    B, L = indices.shape
    V, D = emb_table.shape
    NW = _NC * _NS
    BPW = B // NW
    NCD = D // _LANES

    mesh = plsc.VectorSubcoreMesh(
        core_axis_name="c", subcore_axis_name="s",
        num_cores=_NC, num_subcores=_NS)

    LG = (L + 7) // 8 * 8
    NBUF = 4

    @functools.partial(
        pl.kernel,
        mesh=mesh,
        out_type=jax.ShapeDtypeStruct((B, D), jnp.float32),
        scratch_types=[
            pltpu.VMEM((BPW, 128), jnp.int32),
            pltpu.VMEM((NBUF, LG, D), jnp.float32),
            pltpu.VMEM((BPW, D), jnp.float32),
            [pltpu.SemaphoreType.DMA] * NBUF,
        ],
        compiler_params=pltpu.CompilerParams(use_tc_tiling_on_sc=False),
    )
    def cbow_kernel(idx_hbm, table_hbm, out_hbm, idx_v, rows_v, acc_v, sems):
        wid = lax.axis_index("s") * _NC + lax.axis_index("c")
        pltpu.sync_copy(idx_hbm.at[pl.ds(wid * BPW, BPW)], idx_v)

        def start(j, b):
            pltpu.async_copy(table_hbm.at[idx_v.at[j, pl.ds(0, LG)]],
                             rows_v.at[b], sems[b])

        def finish(j, b):
            pltpu.make_async_copy(
                table_hbm.at[idx_v.at[j, pl.ds(0, LG)]],
                rows_v.at[b], sems[b]).wait()
            accs = [rows_v[b, 0, pl.ds(c * _LANES, _LANES)]
                    for c in range(NCD)]
            for r in range(1, L):
                for c in range(NCD):
                    accs[c] = accs[c] + rows_v[b, r, pl.ds(c * _LANES, _LANES)]
            for c in range(NCD):
                acc_v[j, pl.ds(c * _LANES, _LANES)] = accs[c]

        for b in range(NBUF):
            start(b, b)

        @pl.loop(0, BPW - NBUF, step=NBUF)
        def chunk(j0):
            for b in range(NBUF):
                finish(j0 + b, b)
                start(j0 + b + NBUF, b)

        for b in range(NBUF):
            finish(BPW - NBUF + b, b)

        pltpu.sync_copy(acc_v, out_hbm.at[pl.ds(wid * BPW, BPW)])

    idx_pad = jnp.pad(indices, ((0, 0), (0, 128 - L)))
    return cbow_kernel(idx_pad, emb_table)


def _ffn_tensorcore(cbow, W1, b1, W2, b2):
    B, D = cbow.shape
    H = W1.shape[1]
    C = W2.shape[1]
    BB = 2048

    def body(x_ref, w1_ref, b1_ref, w2_ref, b2_ref, o_ref):
        h = jnp.dot(x_ref[:], w1_ref[:], preferred_element_type=jnp.float32)
        h = jnp.maximum(h + b1_ref[:], 0.0)
        o_ref[:] = jnp.dot(h, w2_ref[:],
                           preferred_element_type=jnp.float32) + b2_ref[:]

    return pl.pallas_call(
        body,
        grid=(B // BB,),
        in_specs=[
            pl.BlockSpec((BB, D), lambda i: (i, 0)),
            pl.BlockSpec((D, H), lambda i: (0, 0)),
            pl.BlockSpec((1, H), lambda i: (0, 0)),
            pl.BlockSpec((H, C), lambda i: (0, 0)),
            pl.BlockSpec((1, C), lambda i: (0, 0)),
        ],
        out_specs=pl.BlockSpec((BB, C), lambda i: (i, 0)),
        out_shape=jax.ShapeDtypeStruct((B, C), jnp.float32),
    )(cbow, W1, b1.reshape(1, H), W2, b2.reshape(1, C))


def kernel(indices, emb_table, W1, b1, W2, b2):
    cbow = _cbow_sparsecore(indices, emb_table)
    return _ffn_tensorcore(cbow, W1, b1, W2, b2)


# speedup vs baseline: 3.9630x; 3.9630x over previous
import functools

import jax
import jax.numpy as jnp
from jax import lax
from jax.experimental import pallas as pl
from jax.experimental.pallas import tpu as pltpu
from jax.experimental.pallas import tpu_sc as plsc

_NC = 2
_NS = 16
_LANES = 16


def _fold_w1_tensorcore(emb_table, W1):
    V, D = emb_table.shape
    H = W1.shape[1]
    BLK = 8192
    grid = (V + BLK - 1) // BLK

    def body(tT_ref, w1_ref, o_ref):
        o_ref[:] = jax.lax.dot_general(
            tT_ref[:], w1_ref[:], (((0,), (0,)), ((), ())),
            preferred_element_type=jnp.float32)

    return pl.pallas_call(
        body,
        grid=(grid,),
        in_specs=[
            pl.BlockSpec((D, BLK), lambda i: (0, i)),
            pl.BlockSpec((D, H), lambda i: (0, 0)),
        ],
        out_specs=pl.BlockSpec((BLK, H), lambda i: (i, 0)),
        out_shape=jax.ShapeDtypeStruct((V, H), jnp.float32),
    )(emb_table.T, W1)


def _cbow_sparsecore(indices, tablew):
    B, L = indices.shape
    V, H = tablew.shape
    NW = _NC * _NS
    BPW = B // NW
    NCD = H // _LANES

    mesh = plsc.VectorSubcoreMesh(
        core_axis_name="c", subcore_axis_name="s",
        num_cores=_NC, num_subcores=_NS)

    NBUF = 2

    @functools.partial(
        pl.kernel,
        mesh=mesh,
        out_type=jax.ShapeDtypeStruct((B, H), jnp.float32),
        scratch_types=[
            pltpu.VMEM((BPW, L), jnp.int32),
            pltpu.VMEM((NBUF, L, H), jnp.float32),
            pltpu.VMEM((BPW // 2, H), jnp.float32),
            [pltpu.SemaphoreType.DMA] * NBUF,
        ],
        compiler_params=pltpu.CompilerParams(use_tc_tiling_on_sc=False),
    )
    def cbow_kernel(idx_hbm, table_hbm, out_hbm, idx_v, rows_v, acc_v, sems):
        wid = lax.axis_index("s") * _NC + lax.axis_index("c")
        HB = BPW // 2
        pltpu.sync_copy(idx_hbm.at[pl.ds(wid * BPW, BPW)], idx_v)

        def start(j, b):
            pltpu.async_copy(table_hbm.at[idx_v.at[j]], rows_v.at[b], sems[b])

        def finish(j, jout, b):
            pltpu.make_async_copy(
                table_hbm.at[idx_v.at[j]], rows_v.at[b], sems[b]).wait()
            for cg in range(0, NCD, 4):
                accs = [rows_v[b, 0, pl.ds((cg + c) * _LANES, _LANES)]
                        for c in range(4)]
                for r in range(1, L):
                    for c in range(4):
                        accs[c] = accs[c] + rows_v[
                            b, r, pl.ds((cg + c) * _LANES, _LANES)]
                for c in range(4):
                    acc_v[jout, pl.ds((cg + c) * _LANES, _LANES)] = accs[c]

        @pl.loop(0, 2)
        def half(hh):
            base = hh * HB
            for b in range(NBUF):
                start(base + b, b)

            @pl.loop(0, HB - NBUF, step=NBUF)
            def chunk(j0):
                for b in range(NBUF):
                    finish(base + j0 + b, j0 + b, b)
                    start(base + j0 + b + NBUF, b)

            for b in range(NBUF):
                finish(base + HB - NBUF + b, HB - NBUF + b, b)

            pltpu.sync_copy(acc_v, out_hbm.at[pl.ds(wid * BPW + base, HB)])

    return cbow_kernel(indices, tablew)


def _ffn_tensorcore(x, b1, W2, b2):
    B, H = x.shape
    C = W2.shape[1]
    BB = 2048

    def body(x_ref, b1_ref, w2_ref, b2_ref, o_ref):
        h = jnp.maximum(x_ref[:] + b1_ref[:], 0.0)
        o_ref[:] = jnp.dot(h, w2_ref[:],
                           preferred_element_type=jnp.float32) + b2_ref[:]

    return pl.pallas_call(
        body,
        grid=(B // BB,),
        in_specs=[
            pl.BlockSpec((BB, H), lambda i: (i, 0)),
            pl.BlockSpec((1, H), lambda i: (0, 0)),
            pl.BlockSpec((H, C), lambda i: (0, 0)),
            pl.BlockSpec((1, C), lambda i: (0, 0)),
        ],
        out_specs=pl.BlockSpec((BB, C), lambda i: (i, 0)),
        out_shape=jax.ShapeDtypeStruct((B, C), jnp.float32),
    )(x, b1.reshape(1, H), W2, b2.reshape(1, C))


def kernel(indices, emb_table, W1, b1, W2, b2):
    tablew = _fold_w1_tensorcore(emb_table, W1)
    hpre = _cbow_sparsecore(indices, tablew)
    return _ffn_tensorcore(hpre, b1, W2, b2)
